# baseline (speedup 1.0000x reference)
import functools

import jax
import jax.numpy as jnp
from jax.experimental import pallas as pl
from jax.experimental.pallas import tpu as pltpu


def _sage_kernel(
    adj_ref,
    adj_hbm_ref,
    x_ref,
    wt1_ref,
    wb1_ref,
    b1_ref,
    wc2_ref,
    bc2_ref,
    o_ref,
    y1_ref,
    hw2_ref,
    y2_ref,
    outp_ref,
    deg_ref,
    tail_ref,
    buf0_ref,
    buf1_ref,
    sem0,
    sem1,
    *,
    bm,
    cw,
    nc,
    tw,
):
    t = pl.program_id(0)
    i = pl.program_id(1)
    n = x_ref.shape[0]
    c = o_ref.shape[1]
    rows = pl.ds(i * bm, bm)

    @pl.when((t == 0) & (i == 0))
    def _():
        y1_ref[...] = jnp.dot(
            x_ref[...], wb1_ref[...], preferred_element_type=jnp.float32
        ).astype(jnp.bfloat16)
        y2_ref[...] = jnp.zeros_like(y2_ref)

    @pl.when(t == 0)
    def _():
        a = adj_ref[...]
        ab = a.astype(jnp.bfloat16)
        deg = jnp.sum(a, axis=1, keepdims=True) + 1e-8
        outp_ref[rows, :] = jnp.dot(
            ab, y2_ref[...], preferred_element_type=jnp.float32
        )
        u = jnp.dot(ab, y1_ref[...], preferred_element_type=jnp.float32)
        h = (
            jnp.dot(x_ref[rows, :], wt1_ref[...], preferred_element_type=jnp.float32)
            + b1_ref[...]
            + u / deg
        )
        p2 = jnp.dot(h, wc2_ref[...], preferred_element_type=jnp.float32) + bc2_ref[...]
        hw2_ref[rows, :] = p2[:, :c]
        y2_ref[rows, :] = p2[:, c:].astype(jnp.bfloat16)
        deg_ref[rows, :] = deg
        tail_ref[rows, :] = ab[:, nc * cw :]

    @pl.when(t == 1)
    def _():
        m1 = (i * bm) // cw
        bufs = (buf0_ref, buf1_ref)
        sems = (sem0, sem1)

        def copy(k):
            return pltpu.make_async_copy(
                adj_hbm_ref.at[pl.ds(i * bm, bm), pl.ds(k * cw, cw)],
                bufs[k % 2],
                sems[k % 2],
            )

        for k in range(nc):
            @pl.when(k == m1)
            def _(k=k):
                copy(k).start()

            if k + 1 < nc:
                @pl.when(k >= m1)
                def _(k=k):
                    copy(k + 1).start()

            @pl.when(k >= m1)
            def _(k=k):
                copy(k).wait()
                ridx = jax.lax.broadcasted_iota(jnp.int32, (cw, c), 0)
                y2s = y2_ref[pl.ds(k * cw, cw), :]
                y2s = jnp.where(ridx + k * cw < i * bm, jnp.bfloat16(0.0), y2s)
                outp_ref[rows, :] += jnp.dot(
                    bufs[k % 2][...].astype(jnp.bfloat16),
                    y2s,
                    preferred_element_type=jnp.float32,
                )

        tidx = jax.lax.broadcasted_iota(jnp.int32, (tw, c), 0)
        y2t = y2_ref[pl.ds(nc * cw, tw), :]
        y2t = jnp.where(tidx + nc * cw < i * bm, jnp.bfloat16(0.0), y2t)
        s = outp_ref[rows, :] + jnp.dot(
            tail_ref[rows, :], y2t, preferred_element_type=jnp.float32
        )
        o_ref[...] = jax.nn.sigmoid(hw2_ref[rows, :] + s / deg_ref[rows, :])


def kernel(x, adj, W1, b1, W2, b2):
    n, f = x.shape
    h1 = W1.shape[1]
    c = W2.shape[1]
    bm = 80 if n % 80 == 0 else n
    ni = n // bm
    cw = 1664
    nc = n // cw
    tw = n - nc * cw

    wt1 = W1[:f]
    wb1 = W1[f:]
    wc2 = jnp.concatenate([W2[:h1], W2[h1:]], axis=1)
    bc2 = jnp.concatenate([b2, jnp.zeros_like(b2)]).reshape(1, 2 * c)

    body = functools.partial(_sage_kernel, bm=bm, cw=cw, nc=nc, tw=tw)
    return pl.pallas_call(
        body,
        grid=(2, ni),
        in_specs=[
            pl.BlockSpec((bm, n), lambda t, i: (jnp.where(t == 0, i, ni - 1), 0)),
            pl.BlockSpec(memory_space=pl.ANY),
            pl.BlockSpec((n, f), lambda t, i: (0, 0)),
            pl.BlockSpec((f, h1), lambda t, i: (0, 0)),
            pl.BlockSpec((f, h1), lambda t, i: (0, 0)),
            pl.BlockSpec((1, h1), lambda t, i: (0, 0)),
            pl.BlockSpec((h1, 2 * c), lambda t, i: (0, 0)),
            pl.BlockSpec((1, 2 * c), lambda t, i: (0, 0)),
        ],
        out_specs=pl.BlockSpec(
            (bm, c), lambda t, i: (jnp.where(t == 1, i, 0), 0)
        ),
        out_shape=jax.ShapeDtypeStruct((n, c), jnp.float32),
        scratch_shapes=[
            pltpu.VMEM((n, h1), jnp.bfloat16),
            pltpu.VMEM((n, c), jnp.float32),
            pltpu.VMEM((n, c), jnp.bfloat16),
            pltpu.VMEM((n, c), jnp.float32),
            pltpu.VMEM((n, 1), jnp.float32),
            pltpu.VMEM((n, tw), jnp.bfloat16),
            pltpu.VMEM((bm, cw), jnp.float32),
            pltpu.VMEM((bm, cw), jnp.float32),
            pltpu.SemaphoreType.DMA,
            pltpu.SemaphoreType.DMA,
        ],
        compiler_params=pltpu.CompilerParams(
            dimension_semantics=("arbitrary", "arbitrary"),
        ),
    )(adj, adj, x, wt1, wb1, b1.reshape(1, h1), wc2, bc2)


# speedup vs baseline: 1.6811x; 1.6811x over previous
import functools

import jax
import jax.numpy as jnp
from jax.experimental import pallas as pl
from jax.experimental.pallas import tpu as pltpu


def _sage_kernel(
    adj_ref,
    adj_hbm_ref,
    x_ref,
    wt1_ref,
    wb1_ref,
    b1_ref,
    wc2_ref,
    bc2_ref,
    o_ref,
    y1_ref,
    hw2_ref,
    y2_ref,
    y2q_ref,
    outp_ref,
    deg_ref,
    tail_ref,
    buf0_ref,
    buf1_ref,
    sem0,
    sem1,
    *,
    bm,
    gb,
    cw,
    nc,
    tw,
):
    t = pl.program_id(0)
    i = pl.program_id(1)
    n = x_ref.shape[0]
    c = o_ref.shape[1]
    rows = pl.ds(i * bm, bm)
    gr = gb // bm

    @pl.when((t == 0) & (i == 0))
    def _():
        y1_ref[...] = jnp.dot(
            x_ref[...], wb1_ref[...], preferred_element_type=jnp.float32
        ).astype(jnp.bfloat16)
        y2q_ref[...] = jnp.zeros_like(y2q_ref)

    @pl.when(t == 0)
    def _():
        @pl.when((i % gr == 0) & (i > 0))
        def _():
            grp = pl.ds((i // gr - 1) * gb, gb)
            y2q_ref[grp, :] = y2_ref[grp, :]

        a = adj_ref[...]
        ab = a.astype(jnp.bfloat16)
        deg = jnp.sum(a, axis=1, keepdims=True) + 1e-8
        outp_ref[rows, :] = jnp.dot(
            ab, y2q_ref[...], preferred_element_type=jnp.float32
        )
        u = jnp.dot(ab, y1_ref[...], preferred_element_type=jnp.float32)
        h = (
            jnp.dot(x_ref[rows, :], wt1_ref[...], preferred_element_type=jnp.float32)
            + b1_ref[...]
            + u / deg
        )
        p2 = jnp.dot(h, wc2_ref[...], preferred_element_type=jnp.float32) + bc2_ref[...]
        hw2_ref[rows, :] = p2[:, :c]
        y2_ref[rows, :] = p2[:, c:].astype(jnp.bfloat16)
        deg_ref[rows, :] = deg
        tail_ref[rows, :] = ab[:, nc * cw :]

    @pl.when((t == 1) & (i % gr == 0))
    def _():
        g = i // gr
        grows = pl.ds(g * gb, gb)
        m1 = (g * gb) // cw
        bufs = (buf0_ref, buf1_ref)
        sems = (sem0, sem1)

        def copy(k):
            return pltpu.make_async_copy(
                adj_hbm_ref.at[pl.ds(g * gb, gb), pl.ds(k * cw, cw)],
                bufs[k % 2],
                sems[k % 2],
            )

        for k in range(nc):
            @pl.when(k == m1)
            def _(k=k):
                copy(k).start()

            if k + 1 < nc:
                @pl.when(k >= m1)
                def _(k=k):
                    copy(k + 1).start()

            @pl.when(k >= m1)
            def _(k=k):
                copy(k).wait()
                ridx = jax.lax.broadcasted_iota(jnp.int32, (cw, c), 0)
                y2s = y2_ref[pl.ds(k * cw, cw), :]
                y2s = jnp.where(ridx + k * cw < g * gb, jnp.bfloat16(0.0), y2s)
                outp_ref[grows, :] += jnp.dot(
                    bufs[k % 2][...].astype(jnp.bfloat16),
                    y2s,
                    preferred_element_type=jnp.float32,
                )

        tidx = jax.lax.broadcasted_iota(jnp.int32, (tw, c), 0)
        y2t = y2_ref[pl.ds(nc * cw, tw), :]
        y2t = jnp.where(tidx + nc * cw < g * gb, jnp.bfloat16(0.0), y2t)
        s = outp_ref[grows, :] + jnp.dot(
            tail_ref[grows, :], y2t, preferred_element_type=jnp.float32
        )
        o_ref[...] = jax.nn.sigmoid(hw2_ref[grows, :] + s / deg_ref[grows, :])


def kernel(x, adj, W1, b1, W2, b2):
    n, f = x.shape
    h1 = W1.shape[1]
    c = W2.shape[1]
    bm = 80 if n % 80 == 0 else n
    gb = 400 if (n % 400 == 0 and bm == 80) else bm
    gr = gb // bm
    ni = n // bm
    cw = 1664
    nc = n // cw
    tw = n - nc * cw

    wt1 = W1[:f]
    wb1 = W1[f:]
    wc2 = jnp.concatenate([W2[:h1], W2[h1:]], axis=1)
    bc2 = jnp.concatenate([b2, jnp.zeros_like(b2)]).reshape(1, 2 * c)

    body = functools.partial(_sage_kernel, bm=bm, gb=gb, cw=cw, nc=nc, tw=tw)
    return pl.pallas_call(
        body,
        grid=(2, ni),
        in_specs=[
            pl.BlockSpec((bm, n), lambda t, i: (jnp.where(t == 0, i, ni - 1), 0)),
            pl.BlockSpec(memory_space=pl.ANY),
            pl.BlockSpec((n, f), lambda t, i: (0, 0)),
            pl.BlockSpec((f, h1), lambda t, i: (0, 0)),
            pl.BlockSpec((f, h1), lambda t, i: (0, 0)),
            pl.BlockSpec((1, h1), lambda t, i: (0, 0)),
            pl.BlockSpec((h1, 2 * c), lambda t, i: (0, 0)),
            pl.BlockSpec((1, 2 * c), lambda t, i: (0, 0)),
        ],
        out_specs=pl.BlockSpec(
            (gb, c), lambda t, i: (jnp.where(t == 1, i // gr, 0), 0)
        ),
        out_shape=jax.ShapeDtypeStruct((n, c), jnp.float32),
        scratch_shapes=[
            pltpu.VMEM((n, h1), jnp.bfloat16),
            pltpu.VMEM((n, c), jnp.float32),
            pltpu.VMEM((n, c), jnp.bfloat16),
            pltpu.VMEM((n, c), jnp.bfloat16),
            pltpu.VMEM((n, c), jnp.float32),
            pltpu.VMEM((n, 1), jnp.float32),
            pltpu.VMEM((n, tw), jnp.bfloat16),
            pltpu.VMEM((gb, cw), jnp.float32),
            pltpu.VMEM((gb, cw), jnp.float32),
            pltpu.SemaphoreType.DMA,
            pltpu.SemaphoreType.DMA,
        ],
        compiler_params=pltpu.CompilerParams(
            dimension_semantics=("arbitrary", "arbitrary"),
        ),
    )(adj, adj, x, wt1, wb1, b1.reshape(1, h1), wc2, bc2)


# speedup vs baseline: 2.3608x; 1.4043x over previous
import functools

import jax
import jax.numpy as jnp
from jax.experimental import pallas as pl
from jax.experimental.pallas import tpu as pltpu


def _sage_kernel(
    adj_ref,
    x_ref,
    wt1_ref,
    wb1_ref,
    b1_ref,
    wc2_ref,
    bc2_ref,
    o_ref,
    y1_ref,
    hw2_ref,
    y2_ref,
):
    t = pl.program_id(0)
    i = pl.program_id(1)
    bm = adj_ref.shape[0]
    c = o_ref.shape[1]

    @pl.when((t == 0) & (i == 0))
    def _():
        y1_ref[...] = jnp.dot(
            x_ref[...], wb1_ref[...], preferred_element_type=jnp.float32
        ).astype(jnp.bfloat16)

    a = adj_ref[...]
    deg = jnp.sum(a, axis=1, keepdims=True) + 1e-8
    ab = a.astype(jnp.bfloat16)
    rows = pl.ds(i * bm, bm)

    @pl.when(t == 0)
    def _():
        agg = jnp.dot(ab, y1_ref[...], preferred_element_type=jnp.float32) / deg
        h = (
            jnp.dot(x_ref[rows, :], wt1_ref[...], preferred_element_type=jnp.float32)
            + b1_ref[...]
            + agg
        )
        p2 = (
            jnp.dot(h, wc2_ref[...], preferred_element_type=jnp.float32)
            + bc2_ref[...]
        )
        hw2_ref[rows, :] = p2[:, :c]
        y2_ref[rows, :] = p2[:, c:].astype(jnp.bfloat16)

    @pl.when(t == 1)
    def _():
        agg = jnp.dot(ab, y2_ref[...], preferred_element_type=jnp.float32) / deg
        o_ref[...] = jax.nn.sigmoid(hw2_ref[rows, :] + agg)


def kernel(x, adj, W1, b1, W2, b2):
    n, f = x.shape
    h1 = W1.shape[1]
    c = W2.shape[1]
    bm = 400 if n % 400 == 0 else n

    wt1 = W1[:f]
    wb1 = W1[f:]
    wc2 = jnp.concatenate([W2[:h1], W2[h1:]], axis=1)
    bc2 = jnp.concatenate([b2, jnp.zeros_like(b2)]).reshape(1, 2 * c)

    return pl.pallas_call(
        _sage_kernel,
        grid=(2, n // bm),
        in_specs=[
            pl.BlockSpec((bm, n), lambda t, i: (i, 0)),
            pl.BlockSpec((n, f), lambda t, i: (0, 0)),
            pl.BlockSpec((f, h1), lambda t, i: (0, 0)),
            pl.BlockSpec((f, h1), lambda t, i: (0, 0)),
            pl.BlockSpec((1, h1), lambda t, i: (0, 0)),
            pl.BlockSpec((h1, 2 * c), lambda t, i: (0, 0)),
            pl.BlockSpec((1, 2 * c), lambda t, i: (0, 0)),
        ],
        out_specs=pl.BlockSpec((bm, c), lambda t, i: (i, 0)),
        out_shape=jax.ShapeDtypeStruct((n, c), jnp.float32),
        scratch_shapes=[
            pltpu.VMEM((n, h1), jnp.bfloat16),
            pltpu.VMEM((n, c), jnp.float32),
            pltpu.VMEM((n, c), jnp.bfloat16),
        ],
        compiler_params=pltpu.CompilerParams(
            dimension_semantics=("arbitrary", "arbitrary"),
        ),
    )(adj, x, wt1, wb1, b1.reshape(1, h1), wc2, bc2)
